# baseline (speedup 1.0000x reference)
import functools

import jax
import jax.numpy as jnp
from jax import lax
from jax.experimental import pallas as pl
from jax.experimental.pallas import tpu as pltpu
from jax.experimental.pallas import tpu_sc as plsc
from jax._src.pallas import mpmd as _mpmd

_BETA_CE = 10.0
_BETA_CU = 10.0
_MA_WINDOW = 100
_CAPACITY = 0.5


def _signal(d_st, d_ch, raw_o_ce, raw_m_cu):
    B, T = d_st.shape
    o_ce_pos = jax.nn.softplus(raw_o_ce)
    m_cu_pos = jax.nn.softplus(raw_m_cu)
    CE = d_st - (d_ch - jnp.log(o_ce_pos + 1e-10))
    W = min(_MA_WINDOW, T)
    if W <= 1:
        ma = d_st
    else:
        pad = jnp.repeat(d_st[:, :1], W - 1, axis=1)
        padded = jnp.concatenate([pad, d_st], axis=1)
        cs = jnp.cumsum(padded, axis=1)
        cs = jnp.concatenate([jnp.zeros((B, 1), dtype=d_st.dtype), cs], axis=1)
        ma = (cs[:, W:] - cs[:, :-W]) / W
    CU = d_st - m_cu_pos * ma
    S_CE = jax.nn.sigmoid(_BETA_CE * CE)
    S_CU = jax.nn.sigmoid(_BETA_CU * CU)
    return S_CE + S_CU - S_CE * S_CU


def _rank_body(g_ref, rank_ref):
    _, T = g_ref.shape
    CH = 256
    jj = lax.broadcasted_iota(jnp.int32, (CH, T), 1)
    g_row = g_ref[0, :].reshape(1, T)
    for c in range(T // CH):
        vi = g_ref[0, c * CH:(c + 1) * CH].reshape(CH, 1)
        ii = lax.broadcasted_iota(jnp.int32, (CH, 1), 0) + (c * CH)
        before = (g_row > vi) | ((g_row == vi) & (jj < ii))
        cnt = jnp.sum(before.astype(jnp.float32), axis=1)
        rank_ref[0, c * CH:(c + 1) * CH] = cnt.astype(jnp.int32)


def _ranks_row(g_row):
    T = g_row.shape[1]
    return pl.pallas_call(
        _rank_body,
        out_shape=jax.ShapeDtypeStruct((1, T), jnp.int32),
    )(g_row)


def _make_sc_batch(b, B, T, D, K, aliased):
    info = plsc.get_sparse_core_info()
    NC, NS = info.num_cores, info.num_subcores
    NW = NC * NS
    RPW = K // NW
    TROWS = 48
    CH = 8
    NCH = TROWS // CH
    NB = 3
    G = 8
    NSL = 4 * G
    SCHUNK = 512

    vmesh = plsc.VectorSubcoreMesh(core_axis_name="c", subcore_axis_name="s")
    smesh = plsc.ScalarSubcoreMesh(axis_name="c")

    def tec_fn(rank_hbm, g_hbm, hid_hbm, *rest):
        if aliased:
            (buf_hbm, out_hbm, idx_hbm, val_hbm, spbuf) = rest
            del buf_hbm
        else:
            (out_hbm, idx_hbm, val_hbm, spbuf) = rest
        del spbuf

        def inner(rank_v, g_v, perm_v, val_v, src_v, rows_v, *sems):
            gsems, osems = sems[:NB], sems[NB:]
            wid = lax.axis_index("s") * NC + lax.axis_index("c")
            lo = pl.multiple_of(wid * RPW, RPW)
            qbase = b * K + lo

            pltpu.sync_copy(rank_hbm, rank_v)
            pltpu.sync_copy(g_hbm, g_v)

            lane = lax.broadcasted_iota(jnp.int32, (16,), 0)

            def scatter_step(t, carry):
                r = rank_v[pl.ds(pl.multiple_of(t * 16, 16), 16)]
                m = (r >= lo) & (r < lo + RPW)
                plsc.store_scatter(perm_v, [r - lo], lane + t * 16, mask=m)
                return carry

            lax.fori_loop(0, T // 16, scatter_step, 0)

            for t in range(RPW // 16):
                p = perm_v[pl.ds(t * 16, 16)]
                val_v[pl.ds(t * 16, 16)] = plsc.load_gather(g_v, [p])
                src_v[pl.ds(t * 16, 16)] = p + b * T

            pltpu.sync_copy(perm_v, idx_hbm.at[pl.ds(lo, RPW)])
            pltpu.sync_copy(val_v, val_hbm.at[pl.ds(lo, RPW)])

            def start_gather(t):
                return pltpu.async_copy(
                    hid_hbm.at[src_v.at[pl.ds(t * CH, CH)]],
                    rows_v.at[t % NB], gsems[t % NB])

            def start_write(t):
                return pltpu.async_copy(
                    rows_v.at[t % NB],
                    out_hbm.at[pl.ds(qbase + t * CH, CH)], osems[t % NB])

            gd, wd = {}, {}
            for u in range(min(NB - 1, NCH)):
                gd[u] = start_gather(u)
            for t in range(NCH):
                gd[t].wait()
                wd[t] = start_write(t)
                nxt = t + NB - 1
                if nxt < NCH:
                    if t >= 1:
                        wd[t - 1].wait()
                        wd[t - 1] = None
                    gd[nxt] = start_gather(nxt)
            for t in range(NCH):
                if wd[t] is not None:
                    wd[t].wait()

        pl.run_scoped(
            inner,
            pltpu.VMEM((T,), jnp.int32),
            pltpu.VMEM((T,), jnp.float32),
            pltpu.VMEM((RPW,), jnp.int32),
            pltpu.VMEM((RPW,), jnp.float32),
            pltpu.VMEM((RPW,), jnp.int32),
            pltpu.VMEM((NB, CH, D), jnp.float32),
            *([pltpu.SemaphoreType.DMA] * (2 * NB)),
        )

    def scs_fn(rank_hbm, g_hbm, hid_hbm, *rest):
        if aliased:
            (buf_hbm, out_hbm, idx_hbm, val_hbm, spbuf) = rest
            del buf_hbm
        else:
            (out_hbm, idx_hbm, val_hbm, spbuf) = rest
        del idx_hbm, val_hbm, g_hbm
        cid = lax.axis_index("c")
        tbase = cid * (T // NC)
        sbase = cid * NSL

        NGRP = NSL // G

        def inner(rank_s, pk_s, dst_s, gsem, osem):
            def drain(sem, nrows):
                pltpu.make_async_copy(
                    hid_hbm.at[pl.ds(0, nrows)],
                    spbuf.at[pl.ds(sbase, nrows)], sem).wait()

            def issue_gathers(g, n_pred):
                slotb = sbase + lax.rem(g, NGRP) * G
                dbase = lax.rem(g, NGRP) * G
                for j in range(G):
                    @pl.when(j < n_pred)
                    def _():
                        pk = pk_s[j]
                        src = b * T + pk // 4096
                        dst_s[dbase + j] = b * K + lax.rem(pk, 4096)
                        pltpu.async_copy(hid_hbm.at[pl.ds(src, 1)],
                                         spbuf.at[pl.ds(slotb + j, 1)], gsem)

            def issue_outs(g, n_pred):
                slotb = sbase + lax.rem(g, NGRP) * G
                dbase = lax.rem(g, NGRP) * G
                for j in range(G):
                    @pl.when(j < n_pred)
                    def _():
                        pltpu.async_copy(
                            spbuf.at[pl.ds(slotb + j, 1)],
                            out_hbm.at[pl.ds(dst_s[dbase + j], 1)], osem)

            def flush(cnt, drained):
                g = cnt // G - 1

                def dr(dv):
                    drain(osem, G)
                    return dv + G
                drained = lax.cond(g >= NGRP, dr, lambda dv: dv, drained)
                issue_gathers(g, G)

                @pl.when(g >= 1)
                def _():
                    drain(gsem, G)
                issue_outs(g - 1, jnp.where(g >= 1, G, 0))
                return drained

            def scan_chunk(c, carry):
                pltpu.sync_copy(
                    rank_hbm.at[pl.ds(tbase + c * SCHUNK, SCHUNK)], rank_s)

                def step(j, carry):
                    cnt, drained = carry
                    r = rank_s[j]
                    take = (r < K) & (lax.rem(r, RPW) >= TROWS)

                    def on_take(carry):
                        cnt, drained = carry
                        i = tbase + c * SCHUNK + j
                        pk_s[lax.rem(cnt, G)] = i * 4096 + r
                        cnt = cnt + 1
                        drained = lax.cond(
                            lax.rem(cnt, G) == 0,
                            lambda dv: flush(cnt, dv),
                            lambda dv: dv, drained)
                        return (cnt, drained)

                    return lax.cond(take, on_take, lambda cr: cr,
                                    (cnt, drained))

                return lax.fori_loop(0, SCHUNK, step, carry, unroll=4)

            cnt, drained = lax.fori_loop(0, T // NC // SCHUNK, scan_chunk,
                                         (jnp.int32(0), jnp.int32(0)))

            gf = cnt // G
            rem = lax.rem(cnt, G)

            def dr2(dv):
                drain(osem, G)
                return dv + G
            drained = lax.cond((rem > 0) & (gf >= NGRP), dr2, lambda dv: dv,
                               drained)
            issue_gathers(gf, rem)

            @pl.when(gf >= 1)
            def _():
                drain(gsem, G)
            issue_outs(gf - 1, jnp.where(gf >= 1, G, 0))

            for j in range(G):
                @pl.when(j < rem)
                def _():
                    drain(gsem, 1)
            issue_outs(gf, rem)

            outstanding = cnt - drained
            for j in range((NGRP + 1) * G):
                @pl.when(j < outstanding)
                def _():
                    drain(osem, 1)

        pl.run_scoped(
            inner,
            pltpu.SMEM((SCHUNK,), jnp.int32),
            pltpu.SMEM((G,), jnp.int32),
            pltpu.SMEM((NSL,), jnp.int32),
            pltpu.SemaphoreType.DMA,
            pltpu.SemaphoreType.DMA,
        )

    out_types = (
        jax.ShapeDtypeStruct((B * K, D), jnp.float32),
        jax.ShapeDtypeStruct((K,), jnp.int32),
        jax.ShapeDtypeStruct((K,), jnp.float32),
    )
    scratch = [
        pltpu.VMEM_SHARED((NC * NSL, D), jnp.float32),
    ]

    return _mpmd._mpmd_map(
        [(smesh, scs_fn), (vmesh, tec_fn)],
        out_types,
        input_output_aliases={3: 0} if aliased else {},
        scratch_types=scratch,
        compiler_params=pltpu.CompilerParams(needs_layout_passes=False),
    )


def kernel(d_st, d_ch, hidden_states, raw_o_ce, raw_m_cu):
    B, T, D = hidden_states.shape
    K = min(max(1, int(T * _CAPACITY)), T)

    g = _signal(d_st, d_ch, raw_o_ce, raw_m_cu)
    hid_flat = hidden_states.reshape(B * T, D)

    buf = None
    idx_parts, val_parts = [], []
    for b in range(B):
        rank_b = _ranks_row(lax.slice(g, (b, 0), (b + 1, T)))
        sc = _make_sc_batch(b, B, T, D, K, aliased=buf is not None)
        args = (rank_b.reshape(T), g[b].reshape(T), hid_flat)
        if buf is not None:
            args = args + (buf,)
        buf, idx_b, val_b = sc(*args)
        idx_parts.append(idx_b)
        val_parts.append(val_b)

    selected = buf
    topk_idx = jnp.concatenate(idx_parts)
    topk_vals = jnp.concatenate(val_parts)
    batch_idx = jnp.repeat(jnp.arange(B, dtype=jnp.int32), K)
    return selected, batch_idx, topk_idx, topk_vals


# speedup vs baseline: 6.4200x; 6.4200x over previous
import functools

import jax
import jax.numpy as jnp
from jax import lax
from jax.experimental import pallas as pl
from jax.experimental.pallas import tpu as pltpu
from jax.experimental.pallas import tpu_sc as plsc
from jax._src.pallas import mpmd as _mpmd

_BETA_CE = 10.0
_BETA_CU = 10.0
_MA_WINDOW = 100
_CAPACITY = 0.5


def _signal(d_st, d_ch, raw_o_ce, raw_m_cu):
    B, T = d_st.shape
    o_ce_pos = jax.nn.softplus(raw_o_ce)
    m_cu_pos = jax.nn.softplus(raw_m_cu)
    CE = d_st - (d_ch - jnp.log(o_ce_pos + 1e-10))
    W = min(_MA_WINDOW, T)
    if W <= 1:
        ma = d_st
    else:
        pad = jnp.repeat(d_st[:, :1], W - 1, axis=1)
        padded = jnp.concatenate([pad, d_st], axis=1)
        cs = jnp.cumsum(padded, axis=1)
        cs = jnp.concatenate([jnp.zeros((B, 1), dtype=d_st.dtype), cs], axis=1)
        ma = (cs[:, W:] - cs[:, :-W]) / W
    CU = d_st - m_cu_pos * ma
    S_CE = jax.nn.sigmoid(_BETA_CE * CE)
    S_CU = jax.nn.sigmoid(_BETA_CU * CU)
    return S_CE + S_CU - S_CE * S_CU


def _rank_body(g_ref, rank_ref):
    _, T = g_ref.shape
    CH = 256
    key = lax.bitcast_convert_type(g_ref[...], jnp.int32)
    kk = key * 2
    kkp = kk + 1
    jj = lax.broadcasted_iota(jnp.int32, (CH, CH), 1)
    ii = lax.broadcasted_iota(jnp.int32, (CH, CH), 0)
    diag_mask = jj < ii
    for c in range(T // CH):
        lo = c * CH
        kki = kk[0, lo:lo + CH].reshape(CH, 1)
        cnt = jnp.zeros((CH,), jnp.float32)
        if lo > 0:
            cnt = cnt + jnp.sum(
                (kkp[:, :lo] > kki).astype(jnp.float32), axis=1)
        if lo + CH < T:
            cnt = cnt + jnp.sum(
                (kk[:, lo + CH:] > kki).astype(jnp.float32), axis=1)
        kkd = kk[0, lo:lo + CH].reshape(1, CH)
        sel = jnp.where(diag_mask, kkd + 1, kkd)
        cnt = cnt + jnp.sum((sel > kki).astype(jnp.float32), axis=1)
        rank_ref[0, lo:lo + CH] = cnt.astype(jnp.int32)


def _ranks_row(g_row):
    T = g_row.shape[1]
    return pl.pallas_call(
        _rank_body,
        out_shape=jax.ShapeDtypeStruct((1, T), jnp.int32),
    )(g_row)


def _make_sc_call(bs, B, T, D, K, aliased):
    info = plsc.get_sparse_core_info()
    NC, NS = info.num_cores, info.num_subcores
    NW = NC * NS
    RPW = K // NW
    CH = 8
    NCHB = RPW // CH
    NB = 3
    NBAT = len(bs)
    NCH = NCHB * NBAT

    mesh = plsc.VectorSubcoreMesh(core_axis_name="c", subcore_axis_name="s")

    def body(rank_hbm, g_hbm, hid_hbm, *rest):
        if aliased:
            (buf_hbm, out_hbm, idx_hbm, val_hbm, *rest) = rest
            del buf_hbm
        else:
            (out_hbm, idx_hbm, val_hbm, *rest) = rest
        rank_v = rest[0:NBAT]
        g_v = rest[NBAT:2 * NBAT]
        perm_v = rest[2 * NBAT:3 * NBAT]
        val_v = rest[3 * NBAT:4 * NBAT]
        src_v = rest[4 * NBAT:5 * NBAT]
        rows_v = rest[5 * NBAT]
        sems = rest[5 * NBAT + 1:]
        gsems, osems = sems[:NB], sems[NB:]

        wid = lax.axis_index("s") * NC + lax.axis_index("c")
        lo = pl.multiple_of(wid * RPW, RPW)
        lane = lax.broadcasted_iota(jnp.int32, (16,), 0)

        for n in range(NBAT):
            pltpu.sync_copy(rank_hbm.at[pl.ds(n * T, T)], rank_v[n])
            pltpu.sync_copy(g_hbm.at[pl.ds(n * T, T)], g_v[n])

        def scan_batch(n):
            def scatter_step(t, carry):
                r = rank_v[n][pl.ds(pl.multiple_of(t * 16, 16), 16)]
                m = (r >= lo) & (r < lo + RPW)
                plsc.store_scatter(perm_v[n], [r - lo], lane + t * 16,
                                   mask=m)
                return carry

            lax.fori_loop(0, T // 16, scatter_step, 0)
            for t in range(RPW // 16):
                p = perm_v[n][pl.ds(t * 16, 16)]
                val_v[n][pl.ds(t * 16, 16)] = plsc.load_gather(g_v[n], [p])
                src_v[n][pl.ds(t * 16, 16)] = p + bs[n] * T

        def start_gather(t):
            n, tc = t // NCHB, t % NCHB
            return pltpu.async_copy(
                hid_hbm.at[src_v[n].at[pl.ds(tc * CH, CH)]],
                rows_v.at[t % NB], gsems[t % NB])

        def start_write(t):
            n, tc = t // NCHB, t % NCHB
            qbase = bs[n] * K + lo
            return pltpu.async_copy(
                rows_v.at[t % NB],
                out_hbm.at[pl.ds(qbase + tc * CH, CH)], osems[t % NB])

        scan_batch(0)
        gd, wd = {}, {}
        for u in range(min(NB - 1, NCHB)):
            gd[u] = start_gather(u)
        for n in range(1, NBAT):
            scan_batch(n)
        for n in range(NBAT):
            pltpu.sync_copy(perm_v[n], idx_hbm.at[pl.ds(n * K + lo, RPW)])
            pltpu.sync_copy(val_v[n], val_hbm.at[pl.ds(n * K + lo, RPW)])

        for u in range(min(NB - 1, NCHB), min(NB - 1, NCH)):
            gd[u] = start_gather(u)
        for t in range(NCH):
            gd[t].wait()
            wd[t] = start_write(t)
            nxt = t + NB - 1
            if nxt < NCH:
                if t >= 1:
                    wd[t - 1].wait()
                    wd[t - 1] = None
                gd[nxt] = start_gather(nxt)
        for t in range(NCH):
            if wd[t] is not None:
                wd[t].wait()

    out_types = (
        jax.ShapeDtypeStruct((B * K, D), jnp.float32),
        jax.ShapeDtypeStruct((NBAT * K,), jnp.int32),
        jax.ShapeDtypeStruct((NBAT * K,), jnp.float32),
    )
    scratch = (
        [pltpu.VMEM((T,), jnp.int32) for _ in range(NBAT)]
        + [pltpu.VMEM((T,), jnp.float32) for _ in range(NBAT)]
        + [pltpu.VMEM((RPW,), jnp.int32) for _ in range(NBAT)]
        + [pltpu.VMEM((RPW,), jnp.float32) for _ in range(NBAT)]
        + [pltpu.VMEM((RPW,), jnp.int32) for _ in range(NBAT)]
        + [pltpu.VMEM((NB, CH, D), jnp.float32)]
        + [pltpu.SemaphoreType.DMA] * (2 * NB)
    )

    return _mpmd._mpmd_map(
        [(mesh, body)],
        out_types,
        input_output_aliases={3: 0} if aliased else {},
        scratch_types=scratch,
        compiler_params=pltpu.CompilerParams(needs_layout_passes=False),
        name="sc_router_" + "".join(str(x) for x in bs),
    )


def kernel(d_st, d_ch, hidden_states, raw_o_ce, raw_m_cu):
    B, T, D = hidden_states.shape
    K = min(max(1, int(T * _CAPACITY)), T)

    g = _signal(d_st, d_ch, raw_o_ce, raw_m_cu)
    hid_flat = hidden_states.reshape(B * T, D)

    ranks = [_ranks_row(lax.slice(g, (b, 0), (b + 1, T))) for b in range(B)]

    groups = [[0], list(range(1, B))] if B > 1 else [[0]]
    buf = None
    idx_parts, val_parts = [], []
    for bs in groups:
        sc = _make_sc_call(bs, B, T, D, K, aliased=buf is not None)
        rank_in = jnp.concatenate(
            [ranks[b].reshape(T) for b in bs], axis=0)
        g_in = jnp.concatenate([g[b].reshape(T) for b in bs], axis=0)
        args = (rank_in, g_in, hid_flat)
        if buf is not None:
            args = args + (buf,)
        buf, idx_b, val_b = sc(*args)
        idx_parts.append(idx_b)
        val_parts.append(val_b)

    selected = buf
    topk_idx = jnp.concatenate(idx_parts)
    topk_vals = jnp.concatenate(val_parts)
    batch_idx = jnp.repeat(jnp.arange(B, dtype=jnp.int32), K)
    return selected, batch_idx, topk_idx, topk_vals


# speedup vs baseline: 6.6122x; 1.0299x over previous
import functools

import jax
import jax.numpy as jnp
from jax import lax
from jax.experimental import pallas as pl
from jax.experimental.pallas import tpu as pltpu
from jax.experimental.pallas import tpu_sc as plsc
from jax._src.pallas import mpmd as _mpmd

_BETA_CE = 10.0
_BETA_CU = 10.0
_MA_WINDOW = 100
_CAPACITY = 0.5


def _signal(d_st, d_ch, raw_o_ce, raw_m_cu):
    B, T = d_st.shape
    o_ce_pos = jax.nn.softplus(raw_o_ce)
    m_cu_pos = jax.nn.softplus(raw_m_cu)
    CE = d_st - (d_ch - jnp.log(o_ce_pos + 1e-10))
    W = min(_MA_WINDOW, T)
    if W <= 1:
        ma = d_st
    else:
        pad = jnp.repeat(d_st[:, :1], W - 1, axis=1)
        padded = jnp.concatenate([pad, d_st], axis=1)
        cs = jnp.cumsum(padded, axis=1)
        cs = jnp.concatenate([jnp.zeros((B, 1), dtype=d_st.dtype), cs], axis=1)
        ma = (cs[:, W:] - cs[:, :-W]) / W
    CU = d_st - m_cu_pos * ma
    S_CE = jax.nn.sigmoid(_BETA_CE * CE)
    S_CU = jax.nn.sigmoid(_BETA_CU * CU)
    return S_CE + S_CU - S_CE * S_CU


def _rank_body(g_ref, rank_ref):
    _, T = g_ref.shape
    CH = 256
    key = lax.bitcast_convert_type(g_ref[...], jnp.int32)
    kk = key * 2
    kkp = kk + 1
    jj = lax.broadcasted_iota(jnp.int32, (CH, CH), 1)
    ii = lax.broadcasted_iota(jnp.int32, (CH, CH), 0)
    diag_mask = jj < ii
    for c in range(T // CH):
        lo = c * CH
        kki = kk[0, lo:lo + CH].reshape(CH, 1)
        cnt = jnp.zeros((CH,), jnp.float32)
        if lo > 0:
            cnt = cnt + jnp.sum(
                (kkp[:, :lo] > kki).astype(jnp.float32), axis=1)
        if lo + CH < T:
            cnt = cnt + jnp.sum(
                (kk[:, lo + CH:] > kki).astype(jnp.float32), axis=1)
        kkd = kk[0, lo:lo + CH].reshape(1, CH)
        sel = jnp.where(diag_mask, kkd + 1, kkd)
        cnt = cnt + jnp.sum((sel > kki).astype(jnp.float32), axis=1)
        rank_ref[0, lo:lo + CH] = cnt.astype(jnp.int32)


def _ranks_row(g_row):
    T = g_row.shape[1]
    return pl.pallas_call(
        _rank_body,
        out_shape=jax.ShapeDtypeStruct((1, T), jnp.int32),
    )(g_row)


def _make_sc_call(bs, B, T, D, K, aliased):
    info = plsc.get_sparse_core_info()
    NC, NS = info.num_cores, info.num_subcores
    NW = NC * NS
    RPW = K // NW
    CH = 8
    NCHB = RPW // CH
    NB = 3
    NBAT = len(bs)
    NCH = NCHB * NBAT

    mesh = plsc.VectorSubcoreMesh(core_axis_name="c", subcore_axis_name="s")

    def body(*args):
        rank_hbm = args[:NBAT]
        g_hbm, hid_hbm, *rest = args[NBAT:]
        if aliased:
            (bi, ii, vi, out_hbm, idx_hbm, val_hbm, *rest) = rest
            del bi, ii, vi
        else:
            (out_hbm, idx_hbm, val_hbm, *rest) = rest
        rank_v = rest[0:NBAT]
        g_v = rest[NBAT:2 * NBAT]
        perm_v = rest[2 * NBAT:3 * NBAT]
        val_v = rest[3 * NBAT:4 * NBAT]
        src_v = rest[4 * NBAT:5 * NBAT]
        rows_v = rest[5 * NBAT]
        sems = rest[5 * NBAT + 1:]
        gsems, osems = sems[:NB], sems[NB:]

        wid = lax.axis_index("s") * NC + lax.axis_index("c")
        lo = pl.multiple_of(wid * RPW, RPW)
        lane = lax.broadcasted_iota(jnp.int32, (16,), 0)

        for n in range(NBAT):
            pltpu.sync_copy(rank_hbm[n], rank_v[n])
            pltpu.sync_copy(g_hbm.at[pl.ds(bs[n] * T, T)], g_v[n])

        def scan_batch(n):
            def scatter_step(t, carry):
                r = rank_v[n][pl.ds(pl.multiple_of(t * 16, 16), 16)]
                m = (r >= lo) & (r < lo + RPW)
                plsc.store_scatter(perm_v[n], [r - lo], lane + t * 16,
                                   mask=m)
                return carry

            lax.fori_loop(0, T // 16, scatter_step, 0)
            for t in range(RPW // 16):
                p = perm_v[n][pl.ds(t * 16, 16)]
                val_v[n][pl.ds(t * 16, 16)] = plsc.load_gather(g_v[n], [p])
                src_v[n][pl.ds(t * 16, 16)] = p + bs[n] * T

        def start_gather(t):
            n, tc = t // NCHB, t % NCHB
            return pltpu.async_copy(
                hid_hbm.at[src_v[n].at[pl.ds(tc * CH, CH)]],
                rows_v.at[t % NB], gsems[t % NB])

        def start_write(t):
            n, tc = t // NCHB, t % NCHB
            qbase = bs[n] * K + lo
            return pltpu.async_copy(
                rows_v.at[t % NB],
                out_hbm.at[pl.ds(qbase + tc * CH, CH)], osems[t % NB])

        scan_batch(0)
        gd, wd = {}, {}
        for u in range(min(NB - 1, NCHB)):
            gd[u] = start_gather(u)
        for n in range(1, NBAT):
            scan_batch(n)
        for n in range(NBAT):
            pltpu.sync_copy(perm_v[n],
                            idx_hbm.at[pl.ds(bs[n] * K + lo, RPW)])
            pltpu.sync_copy(val_v[n],
                            val_hbm.at[pl.ds(bs[n] * K + lo, RPW)])

        for u in range(min(NB - 1, NCHB), min(NB - 1, NCH)):
            gd[u] = start_gather(u)
        for t in range(NCH):
            gd[t].wait()
            wd[t] = start_write(t)
            nxt = t + NB - 1
            if nxt < NCH:
                if t >= 1:
                    wd[t - 1].wait()
                    wd[t - 1] = None
                gd[nxt] = start_gather(nxt)
        for t in range(NCH):
            if wd[t] is not None:
                wd[t].wait()

    out_types = (
        jax.ShapeDtypeStruct((B * K, D), jnp.float32),
        jax.ShapeDtypeStruct((B * K,), jnp.int32),
        jax.ShapeDtypeStruct((B * K,), jnp.float32),
    )
    scratch = (
        [pltpu.VMEM((T,), jnp.int32) for _ in range(NBAT)]
        + [pltpu.VMEM((T,), jnp.float32) for _ in range(NBAT)]
        + [pltpu.VMEM((RPW,), jnp.int32) for _ in range(NBAT)]
        + [pltpu.VMEM((RPW,), jnp.float32) for _ in range(NBAT)]
        + [pltpu.VMEM((RPW,), jnp.int32) for _ in range(NBAT)]
        + [pltpu.VMEM((NB, CH, D), jnp.float32)]
        + [pltpu.SemaphoreType.DMA] * (2 * NB)
    )

    na = NBAT + 2
    return _mpmd._mpmd_map(
        [(mesh, body)],
        out_types,
        input_output_aliases=(
            {na: 0, na + 1: 1, na + 2: 2} if aliased else {}),
        scratch_types=scratch,
        compiler_params=pltpu.CompilerParams(needs_layout_passes=False),
        name="sc_router_" + "".join(str(x) for x in bs),
    )


def kernel(d_st, d_ch, hidden_states, raw_o_ce, raw_m_cu):
    B, T, D = hidden_states.shape
    K = min(max(1, int(T * _CAPACITY)), T)

    g = _signal(d_st, d_ch, raw_o_ce, raw_m_cu)
    hid_flat = hidden_states.reshape(B * T, D)

    ranks = [_ranks_row(lax.slice(g, (b, 0), (b + 1, T))) for b in range(B)]

    g_flat = g.reshape(B * T)
    groups = [[0], list(range(1, B))] if B > 1 else [[0]]
    bufs = None
    for bs in groups:
        sc = _make_sc_call(bs, B, T, D, K, aliased=bufs is not None)
        args = tuple(ranks[b].reshape(T) for b in bs) + (g_flat, hid_flat)
        if bufs is not None:
            args = args + bufs
        bufs = sc(*args)

    selected, topk_idx, topk_vals = bufs
    batch_idx = jnp.repeat(jnp.arange(B, dtype=jnp.int32), K)
    return selected, batch_idx, topk_idx, topk_vals


# speedup vs baseline: 6.7168x; 1.0158x over previous
import functools

import jax
import jax.numpy as jnp
from jax import lax
from jax.experimental import pallas as pl
from jax.experimental.pallas import tpu as pltpu
from jax.experimental.pallas import tpu_sc as plsc
from jax._src.pallas import mpmd as _mpmd

_BETA_CE = 10.0
_BETA_CU = 10.0
_MA_WINDOW = 100
_CAPACITY = 0.5


def _signal(d_st, d_ch, raw_o_ce, raw_m_cu):
    B, T = d_st.shape
    o_ce_pos = jax.nn.softplus(raw_o_ce)
    m_cu_pos = jax.nn.softplus(raw_m_cu)
    CE = d_st - (d_ch - jnp.log(o_ce_pos + 1e-10))
    W = min(_MA_WINDOW, T)
    if W <= 1:
        ma = d_st
    else:
        pad = jnp.repeat(d_st[:, :1], W - 1, axis=1)
        padded = jnp.concatenate([pad, d_st], axis=1)
        cs = jnp.cumsum(padded, axis=1)
        cs = jnp.concatenate([jnp.zeros((B, 1), dtype=d_st.dtype), cs], axis=1)
        ma = (cs[:, W:] - cs[:, :-W]) / W
    CU = d_st - m_cu_pos * ma
    S_CE = jax.nn.sigmoid(_BETA_CE * CE)
    S_CU = jax.nn.sigmoid(_BETA_CU * CU)
    return S_CE + S_CU - S_CE * S_CU


def _rank_body(g_ref, rank_ref):
    NR, T = g_ref.shape
    CH = 256
    keys = lax.bitcast_convert_type(g_ref[...], jnp.int32)
    jj = lax.broadcasted_iota(jnp.int32, (CH, CH), 1)
    ii = lax.broadcasted_iota(jnp.int32, (CH, CH), 0)
    diag_mask = jj < ii
    for b in range(NR):
        kk = keys[b, :].reshape(1, T) * 2
        kkp = kk + 1
        for c in range(T // CH):
            lo = c * CH
            kki = kk[0, lo:lo + CH].reshape(CH, 1)
            cnt = jnp.zeros((CH,), jnp.float32)
            if lo > 0:
                cnt = cnt + jnp.sum(
                    (kkp[:, :lo] > kki).astype(jnp.float32), axis=1)
            if lo + CH < T:
                cnt = cnt + jnp.sum(
                    (kk[:, lo + CH:] > kki).astype(jnp.float32), axis=1)
            kkd = kk[0, lo:lo + CH].reshape(1, CH)
            sel = jnp.where(diag_mask, kkd + 1, kkd)
            cnt = cnt + jnp.sum((sel > kki).astype(jnp.float32), axis=1)
            rank_ref[b, lo:lo + CH] = cnt.astype(jnp.int32)


def _ranks_rows(g_rows):
    NR, T = g_rows.shape
    return pl.pallas_call(
        _rank_body,
        out_shape=jax.ShapeDtypeStruct((NR, T), jnp.int32),
    )(g_rows)


def _make_sc_call(bs, B, T, D, K, aliased):
    info = plsc.get_sparse_core_info()
    NC, NS = info.num_cores, info.num_subcores
    NW = NC * NS
    RPW = K // NW
    CH = 8
    NCHB = RPW // CH
    NB = 3
    NBAT = len(bs)
    NCH = NCHB * NBAT

    mesh = plsc.VectorSubcoreMesh(core_axis_name="c", subcore_axis_name="s")

    def body(*args):
        rank_hbm = args[:NBAT]
        g_hbm, hid_hbm, *rest = args[NBAT:]
        if aliased:
            (bi, ii, vi, out_hbm, idx_hbm, val_hbm, *rest) = rest
            del bi, ii, vi
        else:
            (out_hbm, idx_hbm, val_hbm, *rest) = rest
        rank_v = rest[0:NBAT]
        g_v = rest[NBAT:2 * NBAT]
        perm_v = rest[2 * NBAT:3 * NBAT]
        val_v = rest[3 * NBAT:4 * NBAT]
        src_v = rest[4 * NBAT:5 * NBAT]
        rows_v = rest[5 * NBAT]
        sems = rest[5 * NBAT + 1:]
        gsems, osems = sems[:NB], sems[NB:]

        wid = lax.axis_index("s") * NC + lax.axis_index("c")
        lo = pl.multiple_of(wid * RPW, RPW)
        lane = lax.broadcasted_iota(jnp.int32, (16,), 0)

        for n in range(NBAT):
            pltpu.sync_copy(rank_hbm[n], rank_v[n])
            pltpu.sync_copy(g_hbm.at[pl.ds(bs[n] * T, T)], g_v[n])

        def scan_batch(n):
            def scatter_step(t, carry):
                r = rank_v[n][pl.ds(pl.multiple_of(t * 16, 16), 16)]
                m = (r >= lo) & (r < lo + RPW)
                plsc.store_scatter(perm_v[n], [r - lo], lane + t * 16,
                                   mask=m)
                return carry

            lax.fori_loop(0, T // 16, scatter_step, 0)
            for t in range(RPW // 16):
                p = perm_v[n][pl.ds(t * 16, 16)]
                val_v[n][pl.ds(t * 16, 16)] = plsc.load_gather(g_v[n], [p])
                src_v[n][pl.ds(t * 16, 16)] = p + bs[n] * T

        def start_gather(t):
            n, tc = t // NCHB, t % NCHB
            return pltpu.async_copy(
                hid_hbm.at[src_v[n].at[pl.ds(tc * CH, CH)]],
                rows_v.at[t % NB], gsems[t % NB])

        def start_write(t):
            n, tc = t // NCHB, t % NCHB
            qbase = bs[n] * K + lo
            return pltpu.async_copy(
                rows_v.at[t % NB],
                out_hbm.at[pl.ds(qbase + tc * CH, CH)], osems[t % NB])

        scan_batch(0)
        gd, wd = {}, {}
        for u in range(min(NB - 1, NCHB)):
            gd[u] = start_gather(u)
        for n in range(1, NBAT):
            scan_batch(n)
        for n in range(NBAT):
            pltpu.sync_copy(perm_v[n],
                            idx_hbm.at[pl.ds(bs[n] * K + lo, RPW)])
            pltpu.sync_copy(val_v[n],
                            val_hbm.at[pl.ds(bs[n] * K + lo, RPW)])

        for u in range(min(NB - 1, NCHB), min(NB - 1, NCH)):
            gd[u] = start_gather(u)
        for t in range(NCH):
            gd[t].wait()
            wd[t] = start_write(t)
            nxt = t + NB - 1
            if nxt < NCH:
                if t >= 1:
                    wd[t - 1].wait()
                    wd[t - 1] = None
                gd[nxt] = start_gather(nxt)
        for t in range(NCH):
            if wd[t] is not None:
                wd[t].wait()

    out_types = (
        jax.ShapeDtypeStruct((B * K, D), jnp.float32),
        jax.ShapeDtypeStruct((B * K,), jnp.int32),
        jax.ShapeDtypeStruct((B * K,), jnp.float32),
    )
    scratch = (
        [pltpu.VMEM((T,), jnp.int32) for _ in range(NBAT)]
        + [pltpu.VMEM((T,), jnp.float32) for _ in range(NBAT)]
        + [pltpu.VMEM((RPW,), jnp.int32) for _ in range(NBAT)]
        + [pltpu.VMEM((RPW,), jnp.float32) for _ in range(NBAT)]
        + [pltpu.VMEM((RPW,), jnp.int32) for _ in range(NBAT)]
        + [pltpu.VMEM((NB, CH, D), jnp.float32)]
        + [pltpu.SemaphoreType.DMA] * (2 * NB)
    )

    na = NBAT + 2
    return _mpmd._mpmd_map(
        [(mesh, body)],
        out_types,
        input_output_aliases=(
            {na: 0, na + 1: 1, na + 2: 2} if aliased else {}),
        scratch_types=scratch,
        compiler_params=pltpu.CompilerParams(needs_layout_passes=False),
        name="sc_router_" + "".join(str(x) for x in bs),
    )


def kernel(d_st, d_ch, hidden_states, raw_o_ce, raw_m_cu):
    B, T, D = hidden_states.shape
    K = min(max(1, int(T * _CAPACITY)), T)

    g = _signal(d_st, d_ch, raw_o_ce, raw_m_cu)
    hid_flat = hidden_states.reshape(B * T, D)

    rank0 = _ranks_rows(lax.slice(g, (0, 0), (1, T)))
    if B > 1:
        rank_rest = _ranks_rows(lax.slice(g, (1, 0), (B, T)))
        ranks = [rank0[0]] + [rank_rest[b - 1] for b in range(1, B)]
    else:
        ranks = [rank0[0]]

    g_flat = g.reshape(B * T)
    groups = [[0], list(range(1, B))] if B > 1 else [[0]]
    bufs = None
    for bs in groups:
        sc = _make_sc_call(bs, B, T, D, K, aliased=bufs is not None)
        args = tuple(ranks[b].reshape(T) for b in bs) + (g_flat, hid_flat)
        if bufs is not None:
            args = args + bufs
        bufs = sc(*args)

    selected, topk_idx, topk_vals = bufs
    batch_idx = jnp.repeat(jnp.arange(B, dtype=jnp.int32), K)
    return selected, batch_idx, topk_idx, topk_vals
